# Initial kernel scaffold; baseline (speedup 1.0000x reference)
import jax
import jax.numpy as jnp
from jax.experimental import pallas as pl


def kernel(features_omics1, features_omics2, features_omics3, adj, W_enc1, W_enc2, W_enc3, W_dec1, W_dec2, W_dec3, w_omega, u_omega):
    raise NotImplementedError("write your pallas kernel here")


# speedup vs baseline: 2.7066x; 2.7066x over previous
import jax
import jax.numpy as jnp
from jax.experimental import pallas as pl

_BM_ENC = 2000
_BM = 400


def _proj_body(f1, f2, f3, w1, w2, w3, h_out):
    b = jnp.bfloat16
    h1 = jnp.dot(f1[...].astype(b), w1[...], preferred_element_type=jnp.float32)
    h2 = jnp.dot(f2[...].astype(b), w2[...], preferred_element_type=jnp.float32)
    h3 = jnp.dot(f3[...].astype(b), w3[...], preferred_element_type=jnp.float32)
    h_out[...] = jnp.concatenate([h1, h2, h3], axis=1).astype(b)


def _fuse_body(adj, h, womega, urow, emb_out, embbf_out):
    a = adj[...].astype(jnp.bfloat16)
    z = jnp.dot(a, h[...], preferred_element_type=jnp.float32)
    dz = womega.shape[0]
    zs = [z[:, i * dz:(i + 1) * dz] for i in range(3)]
    w = womega[...]
    u = urow[...]
    ss = []
    for zi in zs:
        v = jnp.tanh(jnp.dot(zi, w, preferred_element_type=jnp.float32,
                             precision=jax.lax.Precision.HIGHEST))
        ss.append(jnp.sum(v * u, axis=1, keepdims=True))
    m = jnp.maximum(jnp.maximum(ss[0], ss[1]), ss[2])
    es = [jnp.exp(s - m) for s in ss]
    den = es[0] + es[1] + es[2]
    emb = (es[0] * zs[0] + es[1] * zs[1] + es[2] * zs[2]) / den
    emb_out[...] = emb
    embbf_out[...] = emb.astype(jnp.bfloat16)


def _dec_body(adj, embbf, wd1, wd2, wd3, r1, r2, r3):
    a = adj[...].astype(jnp.bfloat16)
    ae = jnp.dot(a, embbf[...], preferred_element_type=jnp.float32)
    aeb = ae.astype(jnp.bfloat16)
    r1[...] = jnp.dot(aeb, wd1[...], preferred_element_type=jnp.float32)
    r2[...] = jnp.dot(aeb, wd2[...], preferred_element_type=jnp.float32)
    r3[...] = jnp.dot(aeb, wd3[...], preferred_element_type=jnp.float32)


def kernel(features_omics1, features_omics2, features_omics3, adj,
           W_enc1, W_enc2, W_enc3, W_dec1, W_dec2, W_dec3,
           w_omega, u_omega):
    n, d1 = features_omics1.shape
    d2 = features_omics2.shape[1]
    d3 = features_omics3.shape[1]
    dz = W_enc1.shape[1]
    b = jnp.bfloat16
    w1b, w2b, w3b = W_enc1.astype(b), W_enc2.astype(b), W_enc3.astype(b)
    wd1b, wd2b, wd3b = W_dec1.astype(b), W_dec2.astype(b), W_dec3.astype(b)
    urow = u_omega.reshape(1, dz)

    h = pl.pallas_call(
        _proj_body,
        grid=(n // _BM_ENC,),
        in_specs=[pl.BlockSpec((_BM_ENC, d1), lambda i: (i, 0)),
                  pl.BlockSpec((_BM_ENC, d2), lambda i: (i, 0)),
                  pl.BlockSpec((_BM_ENC, d3), lambda i: (i, 0)),
                  pl.BlockSpec((d1, dz), lambda i: (0, 0)),
                  pl.BlockSpec((d2, dz), lambda i: (0, 0)),
                  pl.BlockSpec((d3, dz), lambda i: (0, 0))],
        out_specs=pl.BlockSpec((_BM_ENC, 3 * dz), lambda i: (i, 0)),
        out_shape=jax.ShapeDtypeStruct((n, 3 * dz), b),
    )(features_omics1, features_omics2, features_omics3, w1b, w2b, w3b)

    emb, embbf = pl.pallas_call(
        _fuse_body,
        grid=(n // _BM,),
        in_specs=[pl.BlockSpec((_BM, n), lambda i: (i, 0)),
                  pl.BlockSpec((n, 3 * dz), lambda i: (0, 0)),
                  pl.BlockSpec((dz, dz), lambda i: (0, 0)),
                  pl.BlockSpec((1, dz), lambda i: (0, 0))],
        out_specs=[pl.BlockSpec((_BM, dz), lambda i: (i, 0)),
                   pl.BlockSpec((_BM, dz), lambda i: (i, 0))],
        out_shape=[jax.ShapeDtypeStruct((n, dz), jnp.float32),
                   jax.ShapeDtypeStruct((n, dz), b)],
    )(adj, h, w_omega, urow)

    r1, r2, r3 = pl.pallas_call(
        _dec_body,
        grid=(n // _BM,),
        in_specs=[pl.BlockSpec((_BM, n), lambda i: (i, 0)),
                  pl.BlockSpec((n, dz), lambda i: (0, 0)),
                  pl.BlockSpec((dz, d1), lambda i: (0, 0)),
                  pl.BlockSpec((dz, d2), lambda i: (0, 0)),
                  pl.BlockSpec((dz, d3), lambda i: (0, 0))],
        out_specs=[pl.BlockSpec((_BM, d1), lambda i: (i, 0)),
                   pl.BlockSpec((_BM, d2), lambda i: (i, 0)),
                   pl.BlockSpec((_BM, d3), lambda i: (i, 0))],
        out_shape=[jax.ShapeDtypeStruct((n, d1), jnp.float32),
                   jax.ShapeDtypeStruct((n, d2), jnp.float32),
                   jax.ShapeDtypeStruct((n, d3), jnp.float32)],
    )(adj, embbf, wd1b, wd2b, wd3b)

    return emb, r1, r2, r3


# speedup vs baseline: 2.7136x; 1.0026x over previous
import jax
import jax.numpy as jnp
from jax.experimental import pallas as pl
from jax.experimental.pallas import tpu as pltpu

_BM = 400


def _body(f1, f2, f3, adj, w1, w2, w3, womega, urow, wd1, wd2, wd3,
          emb_out, r1, r2, r3, h_scr, emb_scr, embbf_scr):
    p = pl.program_id(0)
    i = pl.program_id(1)
    bm = adj.shape[0]
    f32 = jnp.float32
    b16 = jnp.bfloat16

    @pl.when(p == 0)
    def _proj():
        h1 = jnp.dot(f1[...].astype(b16), w1[...], preferred_element_type=f32)
        h2 = jnp.dot(f2[...].astype(b16), w2[...], preferred_element_type=f32)
        h3 = jnp.dot(f3[...].astype(b16), w3[...], preferred_element_type=f32)
        h_scr[pl.ds(i * bm, bm), :] = jnp.concatenate([h1, h2, h3], axis=1).astype(b16)

    @pl.when(p == 1)
    def _fuse():
        a = adj[...].astype(b16)
        z = jnp.dot(a, h_scr[...], preferred_element_type=f32)
        dz = womega.shape[0]
        zs = [z[:, k * dz:(k + 1) * dz] for k in range(3)]
        w = womega[...]
        u = urow[...]
        ss = []
        for zk in zs:
            v = jnp.tanh(jnp.dot(zk.astype(b16), w, preferred_element_type=f32))
            ss.append(jnp.sum(v * u, axis=1, keepdims=True))
        m = jnp.maximum(jnp.maximum(ss[0], ss[1]), ss[2])
        es = [jnp.exp(s - m) for s in ss]
        den = es[0] + es[1] + es[2]
        emb = (es[0] * zs[0] + es[1] * zs[1] + es[2] * zs[2]) / den
        emb_scr[pl.ds(i * bm, bm), :] = emb
        embbf_scr[pl.ds(i * bm, bm), :] = emb.astype(b16)

    @pl.when(p == 2)
    def _dec():
        a = adj[...].astype(b16)
        ae = jnp.dot(a, embbf_scr[...], preferred_element_type=f32)
        aeb = ae.astype(b16)
        r1[...] = jnp.dot(aeb, wd1[...], preferred_element_type=f32)
        r2[...] = jnp.dot(aeb, wd2[...], preferred_element_type=f32)
        r3[...] = jnp.dot(aeb, wd3[...], preferred_element_type=f32)
        emb_out[...] = emb_scr[pl.ds(i * bm, bm), :]


def kernel(features_omics1, features_omics2, features_omics3, adj,
           W_enc1, W_enc2, W_enc3, W_dec1, W_dec2, W_dec3,
           w_omega, u_omega):
    n, d1 = features_omics1.shape
    d2 = features_omics2.shape[1]
    d3 = features_omics3.shape[1]
    dz = W_enc1.shape[1]
    b16 = jnp.bfloat16
    w1b, w2b, w3b = W_enc1.astype(b16), W_enc2.astype(b16), W_enc3.astype(b16)
    wd1b, wd2b, wd3b = W_dec1.astype(b16), W_dec2.astype(b16), W_dec3.astype(b16)
    wob = w_omega.astype(b16)
    urow = u_omega.reshape(1, dz)
    nblk = n // _BM
    last = nblk - 1

    def mov(p, i):
        return (jnp.where(p == 0, i, last), 0)

    def madj(p, i):
        return (jnp.where(p == 0, 0, i), 0)

    def mout(p, i):
        return (jnp.where(p == 2, i, 0), 0)

    def mconst(p, i):
        return (0, 0)

    emb, r1, r2, r3 = pl.pallas_call(
        _body,
        grid=(3, nblk),
        in_specs=[pl.BlockSpec((_BM, d1), mov),
                  pl.BlockSpec((_BM, d2), mov),
                  pl.BlockSpec((_BM, d3), mov),
                  pl.BlockSpec((_BM, n), madj),
                  pl.BlockSpec((d1, dz), mconst),
                  pl.BlockSpec((d2, dz), mconst),
                  pl.BlockSpec((d3, dz), mconst),
                  pl.BlockSpec((dz, dz), mconst),
                  pl.BlockSpec((1, dz), mconst),
                  pl.BlockSpec((dz, d1), mconst),
                  pl.BlockSpec((dz, d2), mconst),
                  pl.BlockSpec((dz, d3), mconst)],
        out_specs=[pl.BlockSpec((_BM, dz), mout),
                   pl.BlockSpec((_BM, d1), mout),
                   pl.BlockSpec((_BM, d2), mout),
                   pl.BlockSpec((_BM, d3), mout)],
        out_shape=[jax.ShapeDtypeStruct((n, dz), jnp.float32),
                   jax.ShapeDtypeStruct((n, d1), jnp.float32),
                   jax.ShapeDtypeStruct((n, d2), jnp.float32),
                   jax.ShapeDtypeStruct((n, d3), jnp.float32)],
        scratch_shapes=[pltpu.VMEM((n, 3 * dz), b16),
                        pltpu.VMEM((n, dz), jnp.float32),
                        pltpu.VMEM((n, dz), b16)],
    )(features_omics1, features_omics2, features_omics3, adj,
      w1b, w2b, w3b, wob, urow, wd1b, wd2b, wd3b)

    return emb, r1, r2, r3
